# baseline (speedup 1.0000x reference)
import functools

import jax
import jax.numpy as jnp
from jax import lax
from jax.experimental import pallas as pl
from jax.experimental.pallas import tpu as pltpu
from jax.experimental.pallas import tpu_sc as plsc

_LOG2 = 0.6931471805599453
_F32 = jnp.float32


def _ssp(t):
    return jnp.maximum(t, 0.0) + jnp.log(1.0 + jnp.exp(-jnp.abs(t))) - _LOG2


def _dot(a, b):
    return jnp.dot(a, b, preferred_element_type=_F32)


def _pre_v_body(x_ref, w1, b1, w2, b2, wdst, wsrc, o_ref, pd_ref, ps_ref):
    h = _ssp(_dot(x_ref[...], w1[...]) + b1[...])
    hx = _ssp(_dot(h, w2[...]) + b2[...])
    o_ref[...] = hx
    pd_ref[...] = _dot(hx, wdst[...])
    ps_ref[...] = _dot(hx, wsrc[...])


def _graph_prep_body(s_ref, w1, b1, w2, b2, wd, be1, vc, bv1,
                     hu_ref, ue_ref, uv_ref):
    h = _ssp(_dot(s_ref[...], w1[...]) + b1[...])
    hu = _ssp(_dot(h, w2[...]) + b2[...])
    hu_ref[...] = hu
    ue_ref[...] = _dot(hu, wd[...]) + be1[...]
    uv_ref[...] = _dot(hu, vc[...]) + bv1[...]


def _edge_body(T, G,
               rd_ref, rs_ref, ea_ref, bb_ref,
               we1, be1, we2, be2, wc, ue, v2, c2, v3, c3,
               e_ref, eo_ref, uacc_ref):
    bf16 = jnp.bfloat16
    t1 = _ssp(_dot(ea_ref[...].astype(bf16), we1[...]) + be1[...])
    he = _ssp(_dot(t1.astype(bf16), we2[...]) + be2[...])
    bb = bb_ref[0, 0, :]
    onehot = (bb[:, None] == lax.broadcasted_iota(jnp.int32, (T, G), 1)
              ).astype(bf16)
    z = (rd_ref[...].astype(_F32) + rs_ref[...].astype(_F32)
         + _dot(he.astype(bf16), wc[...]) + _dot(onehot, ue[...]))
    z1 = _ssp(z)
    z2 = _ssp(_dot(z1.astype(bf16), v2[...]) + c2[...])
    e = _ssp(_dot(z2.astype(bf16), v3[...]) + c3[...])
    e_ref[...] = jnp.concatenate(
        [e, jnp.ones((T, 1), _F32), jnp.zeros((T, 63), _F32)], axis=1)
    eo_ref[...] = e + he

    onehot_t = (bb[None, :] == lax.broadcasted_iota(jnp.int32, (G, T), 0)
                ).astype(_F32)
    eaug = jnp.concatenate([e, jnp.ones((T, 1), _F32)], axis=1)

    @pl.when(pl.program_id(0) == 0)
    def _():
        uacc_ref[...] = jnp.zeros_like(uacc_ref)

    uacc_ref[...] += _dot(onehot_t, eaug)


def _node_body(TN, G, EMB,
               p00_ref, p01_ref, p10_ref, p11_ref, hx_ref, b_ref,
               va, vb, uv, v2, c2, v3, c3,
               xo_ref, uvacc_ref):
    p = (p00_ref[...] + p01_ref[...]) + (p10_ref[...] + p11_ref[...])
    cnt = p[:, EMB:EMB + 1]
    agg = p[:, :EMB] / jnp.maximum(cnt, 1.0)
    b = b_ref[0, 0, :]
    onehot = (b[:, None] == lax.broadcasted_iota(jnp.int32, (TN, G), 1)
              ).astype(_F32)
    hx = hx_ref[...]
    z1 = _ssp(_dot(agg, va[...]) + _dot(hx, vb[...]) + _dot(onehot, uv[...]))
    z2 = _ssp(_dot(z1, v2[...]) + c2[...])
    v = _ssp(_dot(z2, v3[...]) + c3[...])
    xo_ref[...] = v + hx

    onehot_t = (b[None, :] == lax.broadcasted_iota(jnp.int32, (G, TN), 0)
                ).astype(_F32)
    vaug = jnp.concatenate([v, jnp.ones((TN, 1), _F32)], axis=1)

    @pl.when(pl.program_id(0) == 0)
    def _():
        uvacc_ref[...] = jnp.zeros_like(uvacc_ref)

    uvacc_ref[...] += _dot(onehot_t, vaug)


def _final_body(TN, G, EMB,
                x128_ref, b_ref, ueacc_ref, ueacc2_ref, uvacc_ref, hu_ref,
                u1e, u1v, u1h, bu1, u2, bu2, u3, bu3,
                xf_ref, uo_ref):
    @pl.when(pl.program_id(0) == 0)
    def _():
        uea = ueacc_ref[...] + ueacc2_ref[...]
        ue_m = uea[:, :EMB] / jnp.maximum(uea[:, EMB:EMB + 1], 1.0)
        uv_m = uvacc_ref[:, :EMB] / jnp.maximum(uvacc_ref[:, EMB:EMB + 1], 1.0)
        hu = hu_ref[...]
        z1 = _ssp(_dot(ue_m, u1e[...]) + _dot(uv_m, u1v[...])
                  + _dot(hu, u1h[...]) + bu1[...])
        z2 = _ssp(_dot(z1, u2[...]) + bu2[...])
        u = _ssp(_dot(z2, u3[...]) + bu3[...])
        uo_ref[...] = u + hu

    b = b_ref[0, 0, :]
    onehot = (b[:, None] == lax.broadcasted_iota(jnp.int32, (TN, G), 1)
              ).astype(_F32)
    xf_ref[...] = _dot(onehot, x128_ref[...])


_NTILES = 32
_CHUNK = 40
_NBUF = 5


def _sc_gather(pd, ps, dst2, src2, E, D):
    per_tile = dst2.shape[1]
    nchunk = per_tile // _CHUNK
    ngroups = nchunk // _NBUF
    mesh = plsc.VectorSubcoreMesh(core_axis_name="c", subcore_axis_name="s")

    @functools.partial(
        pl.kernel, mesh=mesh,
        out_type=[jax.ShapeDtypeStruct((E, D), _F32),
                  jax.ShapeDtypeStruct((E, D), _F32)],
        scratch_types=([pltpu.VMEM((per_tile,), jnp.int32),
                        pltpu.VMEM((per_tile,), jnp.int32)]
                       + [pltpu.VMEM((_CHUNK, D), _F32)] * (2 * _NBUF)
                       + [pltpu.SemaphoreType.DMA((_NBUF,))] * 4),
    )
    def k(pd_hbm, ps_hbm, dst_hbm, src_hbm, rd_hbm, rs_hbm,
          idx_d, idx_s, *bufs_and_sems):
        bd = bufs_and_sems[:_NBUF]
        bs = bufs_and_sems[_NBUF:2 * _NBUF]
        gsd, gss, wsd, wss = bufs_and_sems[2 * _NBUF:]
        c = lax.axis_index("c")
        s = lax.axis_index("s")
        wid = s * 2 + c
        base = wid * per_tile
        pltpu.sync_copy(dst_hbm.at[wid], idx_d)
        pltpu.sync_copy(src_hbm.at[wid], idx_s)

        def g_d(j, b):
            return pltpu.make_async_copy(
                pd_hbm.at[idx_d.at[pl.ds(j * _CHUNK, _CHUNK)]], bd[b],
                gsd.at[b])

        def g_s(j, b):
            return pltpu.make_async_copy(
                ps_hbm.at[idx_s.at[pl.ds(j * _CHUNK, _CHUNK)]], bs[b],
                gss.at[b])

        def w_d(j, b):
            return pltpu.make_async_copy(
                bd[b], rd_hbm.at[pl.ds(base + j * _CHUNK, _CHUNK)], wsd.at[b])

        def w_s(j, b):
            return pltpu.make_async_copy(
                bs[b], rs_hbm.at[pl.ds(base + j * _CHUNK, _CHUNK)], wss.at[b])

        for b in range(_NBUF):
            g_d(b, b).start()
            g_s(b, b).start()

        def body(g, carry):
            for b in range(_NBUF):
                j = g * _NBUF + b
                g_d(j, b).wait()
                g_s(j, b).wait()
                w_d(j, b).start()
                w_s(j, b).start()
            for b in range(_NBUF):
                j = g * _NBUF + b

                @pl.when(g < ngroups - 1)
                def _(j=j, b=b):
                    w_d(j, b).wait()
                    w_s(j, b).wait()
                    g_d(j + _NBUF, b).start()
                    g_s(j + _NBUF, b).start()
            return carry

        lax.fori_loop(0, ngroups, body, 0)
        jlast = (ngroups - 1) * _NBUF
        for b in range(_NBUF):
            w_d(jlast + b, b).wait()
            w_s(jlast + b, b).wait()

    return k(pd, ps, dst2, src2)


def _sc_scatter(e_aug, dst3, zeros2d, N, D):
    nchunk = dst3.shape[1]
    per_tile = nchunk * _CHUNK
    ngroups = nchunk // _NBUF
    mesh = plsc.VectorSubcoreMesh(core_axis_name="c", subcore_axis_name="s")

    @functools.partial(
        pl.kernel, mesh=mesh,
        out_type=jax.ShapeDtypeStruct((2, N, D), _F32),
        scratch_types=([pltpu.VMEM((_NBUF, _CHUNK), jnp.int32)]
                       + [pltpu.VMEM((_CHUNK, D), _F32)] * _NBUF
                       + [pltpu.SemaphoreType.DMA((_NBUF,))] * 3
                       + [pltpu.VMEM_SHARED((N, D), _F32)]),
    )
    def k(e_hbm, dst_hbm, z2_hbm, aggp_hbm, idx_v, *rest):
        ebuf = rest[:_NBUF]
        rsem, isem, ssem, sh_agg = rest[_NBUF:]
        c = lax.axis_index("c")
        s = lax.axis_index("s")
        wid = s * 2 + c
        base = wid * per_tile

        pltpu.sync_copy(z2_hbm, ebuf[0])
        nzc = N // _CHUNK

        def zbody(t, carry):
            m = s * 16 + t

            @pl.when(m < nzc)
            def _():
                pltpu.sync_copy(ebuf[0], sh_agg.at[pl.ds(m * _CHUNK, _CHUNK)])
            return carry

        lax.fori_loop(0, (nzc + 15) // 16, zbody, 0)
        plsc.subcore_barrier()

        def rd(j, b):
            return pltpu.make_async_copy(
                e_hbm.at[pl.ds(base + j * _CHUNK, _CHUNK)], ebuf[b],
                rsem.at[b])

        def rix(j, b):
            return pltpu.make_async_copy(dst_hbm.at[wid, j], idx_v.at[b],
                                         isem.at[b])

        def sc_wait(b):
            return pltpu.make_async_copy(ebuf[b], sh_agg.at[idx_v.at[b]],
                                         ssem.at[b])

        for b in range(_NBUF):
            rd(b, b).start()
            rix(b, b).start()

        def body(g, carry):
            for b in range(_NBUF):
                j = g * _NBUF + b
                rd(j, b).wait()
                rix(j, b).wait()
                pltpu.async_copy(ebuf[b], sh_agg.at[idx_v.at[b]], ssem.at[b],
                                 add=True)
            for b in range(_NBUF):
                j = g * _NBUF + b

                @pl.when(g < ngroups - 1)
                def _(j=j, b=b):
                    sc_wait(b).wait()
                    rd(j + _NBUF, b).start()
                    rix(j + _NBUF, b).start()
            return carry

        lax.fori_loop(0, ngroups, body, 0)
        for b in range(_NBUF):
            sc_wait(b).wait()

        plsc.subcore_barrier()

        @pl.when(s == 0)
        def _():
            pltpu.sync_copy(sh_agg, aggp_hbm.at[c])

    return k(e_aug, dst3, zeros2d)


def kernel(x, edge_attr, state, params, edge_index, batch, bond_batch):
    N, DN = x.shape
    E, DE = edge_attr.shape
    G, DS = state.shape
    EMB = params['pre_v'][-1][0].shape[1]
    H = 2 * EMB

    (wv1, bv1), (wv2, bv2) = params['pre_v']
    (we1, be1), (we2, be2) = params['pre_e']
    (wu1, bu1), (wu2, bu2) = params['pre_u']
    (pe1, pe1b), (pe2, pe2b), (pe3, pe3b) = params['phi_e']
    (pv1, pv1b), (pv2, pv2b), (pv3, pv3b) = params['phi_v']
    (pu1, pu1b), (pu2, pu2b), (pu3, pu3b) = params['phi_u']

    w_dst, w_src, w_he, w_hub = (pe1[0:EMB], pe1[EMB:2 * EMB],
                                 pe1[2 * EMB:3 * EMB], pe1[3 * EMB:4 * EMB])
    v_agg, v_hx, v_hub = pv1[0:EMB], pv1[EMB:2 * EMB], pv1[2 * EMB:3 * EMB]
    u_ue, u_uv, u_hu = pu1[0:EMB], pu1[EMB:2 * EMB], pu1[2 * EMB:3 * EMB]

    row = lambda v: v.reshape(1, -1)

    TN = 1000
    assert N % TN == 0
    h_x, pd, ps = pl.pallas_call(
        _pre_v_body,
        grid=(N // TN,),
        in_specs=[pl.BlockSpec((TN, DN), lambda i: (i, 0)),
                  pl.BlockSpec((DN, H), lambda i: (0, 0)),
                  pl.BlockSpec((1, H), lambda i: (0, 0)),
                  pl.BlockSpec((H, EMB), lambda i: (0, 0)),
                  pl.BlockSpec((1, EMB), lambda i: (0, 0)),
                  pl.BlockSpec((EMB, H), lambda i: (0, 0)),
                  pl.BlockSpec((EMB, H), lambda i: (0, 0))],
        out_specs=[pl.BlockSpec((TN, EMB), lambda i: (i, 0)),
                   pl.BlockSpec((TN, H), lambda i: (i, 0)),
                   pl.BlockSpec((TN, H), lambda i: (i, 0))],
        out_shape=[jax.ShapeDtypeStruct((N, EMB), _F32),
                   jax.ShapeDtypeStruct((N, H), _F32),
                   jax.ShapeDtypeStruct((N, H), _F32)],
    )(x, wv1, row(bv1), wv2, row(bv2), w_dst, w_src)

    h_u, table_ue, table_uv = pl.pallas_call(
        _graph_prep_body,
        out_shape=[jax.ShapeDtypeStruct((G, EMB), _F32),
                   jax.ShapeDtypeStruct((G, H), _F32),
                   jax.ShapeDtypeStruct((G, H), _F32)],
    )(state, wu1, row(bu1), wu2, row(bu2), w_hub, row(pe1b), v_hub, row(pv1b))

    TE = 2000
    EH = E // 2
    assert EH % (_NTILES * _CHUNK * _NBUF) == 0 and EH % TE == 0
    nchunk = EH // (_NTILES * _CHUNK)
    we1b, we2b = we1.astype(jnp.bfloat16), we2.astype(jnp.bfloat16)
    w_heb = w_he.astype(jnp.bfloat16)
    pe2b_, pe3b_ = pe2.astype(jnp.bfloat16), pe3.astype(jnp.bfloat16)
    e_outs, ueaccs, aggps = [], [], []
    for hh in range(2):
        sl = slice(hh * EH, (hh + 1) * EH)
        dst_h = edge_index[1, sl]
        src_h = edge_index[0, sl]
        dst2 = dst_h.reshape(_NTILES, EH // _NTILES)
        src2 = src_h.reshape(_NTILES, EH // _NTILES)
        dst3 = dst_h.reshape(_NTILES, nchunk, _CHUNK)
        rd, rs = _sc_gather(pd, ps, dst2, src2, EH, H)
        bb3 = bond_batch[sl].reshape(EH // TE, 1, TE)
        e_aug, e_out_h, ueacc_h = pl.pallas_call(
            functools.partial(_edge_body, TE, G),
            grid=(EH // TE,),
            in_specs=[pl.BlockSpec((TE, H), lambda i: (i, 0)),
                      pl.BlockSpec((TE, H), lambda i: (i, 0)),
                      pl.BlockSpec((TE, DE), lambda i: (i, 0)),
                      pl.BlockSpec((1, 1, TE), lambda i: (i, 0, 0)),
                      pl.BlockSpec((DE, H), lambda i: (0, 0)),
                      pl.BlockSpec((1, H), lambda i: (0, 0)),
                      pl.BlockSpec((H, EMB), lambda i: (0, 0)),
                      pl.BlockSpec((1, EMB), lambda i: (0, 0)),
                      pl.BlockSpec((EMB, H), lambda i: (0, 0)),
                      pl.BlockSpec((G, H), lambda i: (0, 0)),
                      pl.BlockSpec((H, H), lambda i: (0, 0)),
                      pl.BlockSpec((1, H), lambda i: (0, 0)),
                      pl.BlockSpec((H, EMB), lambda i: (0, 0)),
                      pl.BlockSpec((1, EMB), lambda i: (0, 0))],
            out_specs=[pl.BlockSpec((TE, H), lambda i: (i, 0)),
                       pl.BlockSpec((TE, EMB), lambda i: (i, 0)),
                       pl.BlockSpec((G, EMB + 1), lambda i: (0, 0))],
            out_shape=[jax.ShapeDtypeStruct((EH, H), _F32),
                       jax.ShapeDtypeStruct((EH, EMB), _F32),
                       jax.ShapeDtypeStruct((G, EMB + 1), _F32)],
        )(rd, rs, edge_attr[sl], bb3,
          we1b, row(be1), we2b, row(be2), w_heb,
          table_ue.astype(jnp.bfloat16),
          pe2b_, row(pe2b), pe3b_, row(pe3b))
        e_outs.append(e_out_h)
        ueaccs.append(ueacc_h)
        aggps.append(_sc_scatter(e_aug, dst3,
                                 jnp.zeros((_CHUNK, H), _F32), N, H))
    e_out = jnp.concatenate(e_outs, axis=0)

    b3 = batch.reshape(N // TN, 1, TN)
    x_out, uvacc = pl.pallas_call(
        functools.partial(_node_body, TN, G, EMB),
        grid=(N // TN,),
        in_specs=[pl.BlockSpec((TN, H), lambda i: (i, 0)),
                  pl.BlockSpec((TN, H), lambda i: (i, 0)),
                  pl.BlockSpec((TN, H), lambda i: (i, 0)),
                  pl.BlockSpec((TN, H), lambda i: (i, 0)),
                  pl.BlockSpec((TN, EMB), lambda i: (i, 0)),
                  pl.BlockSpec((1, 1, TN), lambda i: (i, 0, 0)),
                  pl.BlockSpec((EMB, H), lambda i: (0, 0)),
                  pl.BlockSpec((EMB, H), lambda i: (0, 0)),
                  pl.BlockSpec((G, H), lambda i: (0, 0)),
                  pl.BlockSpec((H, H), lambda i: (0, 0)),
                  pl.BlockSpec((1, H), lambda i: (0, 0)),
                  pl.BlockSpec((H, EMB), lambda i: (0, 0)),
                  pl.BlockSpec((1, EMB), lambda i: (0, 0))],
        out_specs=[pl.BlockSpec((TN, EMB), lambda i: (i, 0)),
                   pl.BlockSpec((G, EMB + 1), lambda i: (0, 0))],
        out_shape=[jax.ShapeDtypeStruct((N, EMB), _F32),
                   jax.ShapeDtypeStruct((G, EMB + 1), _F32)],
    )(aggps[0][0], aggps[0][1], aggps[1][0], aggps[1][1], h_x, b3,
      v_agg, v_hx, table_uv, pv2, row(pv2b), pv3, row(pv3b))

    x_final, u_out = pl.pallas_call(
        functools.partial(_final_body, TN, G, EMB),
        grid=(N // TN,),
        in_specs=[pl.BlockSpec((G, EMB), lambda i: (0, 0)),
                  pl.BlockSpec((1, 1, TN), lambda i: (i, 0, 0)),
                  pl.BlockSpec((G, EMB + 1), lambda i: (0, 0)),
                  pl.BlockSpec((G, EMB + 1), lambda i: (0, 0)),
                  pl.BlockSpec((G, EMB + 1), lambda i: (0, 0)),
                  pl.BlockSpec((G, EMB), lambda i: (0, 0)),
                  pl.BlockSpec((EMB, H), lambda i: (0, 0)),
                  pl.BlockSpec((EMB, H), lambda i: (0, 0)),
                  pl.BlockSpec((EMB, H), lambda i: (0, 0)),
                  pl.BlockSpec((1, H), lambda i: (0, 0)),
                  pl.BlockSpec((H, H), lambda i: (0, 0)),
                  pl.BlockSpec((1, H), lambda i: (0, 0)),
                  pl.BlockSpec((H, EMB), lambda i: (0, 0)),
                  pl.BlockSpec((1, EMB), lambda i: (0, 0))],
        out_specs=[pl.BlockSpec((TN, EMB), lambda i: (i, 0)),
                   pl.BlockSpec((G, EMB), lambda i: (0, 0))],
        out_shape=[jax.ShapeDtypeStruct((N, EMB), _F32),
                   jax.ShapeDtypeStruct((G, EMB), _F32)],
    )(x_out[:G], b3, ueaccs[0], ueaccs[1], uvacc, h_u,
      u_ue, u_uv, u_hu, row(pu1b), pu2, row(pu2b), pu3, row(pu3b))

    return (x_final, e_out, u_out)


# speedup vs baseline: 1.0451x; 1.0451x over previous
import functools

import jax
import jax.numpy as jnp
from jax import lax
from jax.experimental import pallas as pl
from jax.experimental.pallas import tpu as pltpu
from jax.experimental.pallas import tpu_sc as plsc

_LOG2 = 0.6931471805599453
_F32 = jnp.float32


def _ssp(t):
    return jnp.maximum(t, 0.0) + jnp.log(1.0 + jnp.exp(-jnp.abs(t))) - _LOG2


def _dot(a, b):
    return jnp.dot(a, b, preferred_element_type=_F32)


def _pre_v_body(x_ref, w1, b1, w2, b2, wdst, wsrc, o_ref, pd_ref, ps_ref):
    h = _ssp(_dot(x_ref[...], w1[...]) + b1[...])
    hx = _ssp(_dot(h, w2[...]) + b2[...])
    o_ref[...] = hx
    pd_ref[...] = _dot(hx, wdst[...])
    ps_ref[...] = _dot(hx, wsrc[...])


def _graph_prep_body(s_ref, w1, b1, w2, b2, wd, be1, vc, bv1,
                     hu_ref, ue_ref, uv_ref):
    h = _ssp(_dot(s_ref[...], w1[...]) + b1[...])
    hu = _ssp(_dot(h, w2[...]) + b2[...])
    hu_ref[...] = hu
    ue_ref[...] = _dot(hu, wd[...]) + be1[...]
    uv_ref[...] = _dot(hu, vc[...]) + bv1[...]


def _edge_body(T, G, half,
               rd_ref, rs_ref, ea_ref, bb_ref, *rest):
    (we1, be1, we2, be2, wc, ue, v2, c2, v3, c3,
     e_ref, eo_ref, uacc_ref) = rest[1 if half else 0:]
    bf16 = jnp.bfloat16
    t1 = _ssp(_dot(ea_ref[...].astype(bf16), we1[...]) + be1[...])
    he = _ssp(_dot(t1.astype(bf16), we2[...]) + be2[...])
    bb = bb_ref[0, 0, :]
    onehot = (bb[:, None] == lax.broadcasted_iota(jnp.int32, (T, G), 1)
              ).astype(bf16)
    z = (rd_ref[...].astype(_F32) + rs_ref[...].astype(_F32)
         + _dot(he.astype(bf16), wc[...]) + _dot(onehot, ue[...]))
    z1 = _ssp(z)
    z2 = _ssp(_dot(z1.astype(bf16), v2[...]) + c2[...])
    e = _ssp(_dot(z2.astype(bf16), v3[...]) + c3[...])
    e_ref[...] = jnp.concatenate(
        [e, jnp.ones((T, 1), _F32), jnp.zeros((T, 63), _F32)], axis=1)
    eo_ref[...] = e + he

    onehot_t = (bb[None, :] == lax.broadcasted_iota(jnp.int32, (G, T), 0)
                ).astype(_F32)
    eaug = jnp.concatenate([e, jnp.ones((T, 1), _F32)], axis=1)

    @pl.when(pl.program_id(0) == 0)
    def _():
        uacc_ref[...] = jnp.zeros_like(uacc_ref)

    uacc_ref[...] += _dot(onehot_t, eaug)


def _node_body(TN, G, EMB,
               p00_ref, p01_ref, p10_ref, p11_ref, hx_ref, b_ref,
               va, vb, uv, v2, c2, v3, c3,
               xo_ref, uvacc_ref):
    p = (p00_ref[...] + p01_ref[...]) + (p10_ref[...] + p11_ref[...])
    cnt = p[:, EMB:EMB + 1]
    agg = p[:, :EMB] / jnp.maximum(cnt, 1.0)
    b = b_ref[0, 0, :]
    onehot = (b[:, None] == lax.broadcasted_iota(jnp.int32, (TN, G), 1)
              ).astype(_F32)
    hx = hx_ref[...]
    z1 = _ssp(_dot(agg, va[...]) + _dot(hx, vb[...]) + _dot(onehot, uv[...]))
    z2 = _ssp(_dot(z1, v2[...]) + c2[...])
    v = _ssp(_dot(z2, v3[...]) + c3[...])
    xo_ref[...] = v + hx

    onehot_t = (b[None, :] == lax.broadcasted_iota(jnp.int32, (G, TN), 0)
                ).astype(_F32)
    vaug = jnp.concatenate([v, jnp.ones((TN, 1), _F32)], axis=1)

    @pl.when(pl.program_id(0) == 0)
    def _():
        uvacc_ref[...] = jnp.zeros_like(uvacc_ref)

    uvacc_ref[...] += _dot(onehot_t, vaug)


def _final_body(TN, G, EMB,
                x128_ref, b_ref, ueacc_ref, ueacc2_ref, uvacc_ref, hu_ref,
                u1e, u1v, u1h, bu1, u2, bu2, u3, bu3,
                xf_ref, uo_ref):
    @pl.when(pl.program_id(0) == 0)
    def _():
        uea = ueacc_ref[...] + ueacc2_ref[...]
        ue_m = uea[:, :EMB] / jnp.maximum(uea[:, EMB:EMB + 1], 1.0)
        uv_m = uvacc_ref[:, :EMB] / jnp.maximum(uvacc_ref[:, EMB:EMB + 1], 1.0)
        hu = hu_ref[...]
        z1 = _ssp(_dot(ue_m, u1e[...]) + _dot(uv_m, u1v[...])
                  + _dot(hu, u1h[...]) + bu1[...])
        z2 = _ssp(_dot(z1, u2[...]) + bu2[...])
        u = _ssp(_dot(z2, u3[...]) + bu3[...])
        uo_ref[...] = u + hu

    b = b_ref[0, 0, :]
    onehot = (b[:, None] == lax.broadcasted_iota(jnp.int32, (TN, G), 1)
              ).astype(_F32)
    xf_ref[...] = _dot(onehot, x128_ref[...])


_NTILES = 32
_CHUNK = 40
_NBUF = 5


def _sc_gather(pd, ps, dst2, src2, E, D):
    per_tile = dst2.shape[1]
    nchunk = per_tile // _CHUNK
    ngroups = nchunk // _NBUF
    mesh = plsc.VectorSubcoreMesh(core_axis_name="c", subcore_axis_name="s")

    @functools.partial(
        pl.kernel, mesh=mesh,
        out_type=[jax.ShapeDtypeStruct((E, D), _F32),
                  jax.ShapeDtypeStruct((E, D), _F32)],
        scratch_types=([pltpu.VMEM((per_tile,), jnp.int32),
                        pltpu.VMEM((per_tile,), jnp.int32)]
                       + [pltpu.VMEM((_CHUNK, D), _F32)] * (2 * _NBUF)
                       + [pltpu.SemaphoreType.DMA((_NBUF,))] * 4),
    )
    def k(pd_hbm, ps_hbm, dst_hbm, src_hbm, rd_hbm, rs_hbm,
          idx_d, idx_s, *bufs_and_sems):
        bd = bufs_and_sems[:_NBUF]
        bs = bufs_and_sems[_NBUF:2 * _NBUF]
        gsd, gss, wsd, wss = bufs_and_sems[2 * _NBUF:]
        c = lax.axis_index("c")
        s = lax.axis_index("s")
        wid = s * 2 + c
        base = wid * per_tile
        pltpu.sync_copy(dst_hbm.at[wid], idx_d)
        pltpu.sync_copy(src_hbm.at[wid], idx_s)

        def g_d(j, b):
            return pltpu.make_async_copy(
                pd_hbm.at[idx_d.at[pl.ds(j * _CHUNK, _CHUNK)]], bd[b],
                gsd.at[b])

        def g_s(j, b):
            return pltpu.make_async_copy(
                ps_hbm.at[idx_s.at[pl.ds(j * _CHUNK, _CHUNK)]], bs[b],
                gss.at[b])

        def w_d(j, b):
            return pltpu.make_async_copy(
                bd[b], rd_hbm.at[pl.ds(base + j * _CHUNK, _CHUNK)], wsd.at[b])

        def w_s(j, b):
            return pltpu.make_async_copy(
                bs[b], rs_hbm.at[pl.ds(base + j * _CHUNK, _CHUNK)], wss.at[b])

        for b in range(_NBUF):
            g_d(b, b).start()
            g_s(b, b).start()

        def body(g, carry):
            for b in range(_NBUF):
                j = g * _NBUF + b
                g_d(j, b).wait()
                g_s(j, b).wait()
                w_d(j, b).start()
                w_s(j, b).start()
            for b in range(_NBUF):
                j = g * _NBUF + b

                @pl.when(g < ngroups - 1)
                def _(j=j, b=b):
                    w_d(j, b).wait()
                    w_s(j, b).wait()
                    g_d(j + _NBUF, b).start()
                    g_s(j + _NBUF, b).start()
            return carry

        lax.fori_loop(0, ngroups, body, 0)
        jlast = (ngroups - 1) * _NBUF
        for b in range(_NBUF):
            w_d(jlast + b, b).wait()
            w_s(jlast + b, b).wait()

    return k(pd, ps, dst2, src2)


def _sc_scatter(e_aug, dst3, zeros2d, N, D):
    nchunk = dst3.shape[1]
    per_tile = nchunk * _CHUNK
    ngroups = nchunk // _NBUF
    mesh = plsc.VectorSubcoreMesh(core_axis_name="c", subcore_axis_name="s")

    @functools.partial(
        pl.kernel, mesh=mesh,
        out_type=jax.ShapeDtypeStruct((2, N, D), _F32),
        scratch_types=([pltpu.VMEM((_NBUF, _CHUNK), jnp.int32)]
                       + [pltpu.VMEM((_CHUNK, D), _F32)] * _NBUF
                       + [pltpu.SemaphoreType.DMA((_NBUF,))] * 3
                       + [pltpu.VMEM_SHARED((N, D), _F32)]),
    )
    def k(e_hbm, dst_hbm, z2_hbm, aggp_hbm, idx_v, *rest):
        ebuf = rest[:_NBUF]
        rsem, isem, ssem, sh_agg = rest[_NBUF:]
        c = lax.axis_index("c")
        s = lax.axis_index("s")
        wid = s * 2 + c
        base = wid * per_tile

        pltpu.sync_copy(z2_hbm, ebuf[0])
        nzc = N // _CHUNK

        def zbody(t, carry):
            m = s * 16 + t

            @pl.when(m < nzc)
            def _():
                pltpu.sync_copy(ebuf[0], sh_agg.at[pl.ds(m * _CHUNK, _CHUNK)])
            return carry

        lax.fori_loop(0, (nzc + 15) // 16, zbody, 0)
        plsc.subcore_barrier()

        def rd(j, b):
            return pltpu.make_async_copy(
                e_hbm.at[pl.ds(base + j * _CHUNK, _CHUNK)], ebuf[b],
                rsem.at[b])

        def rix(j, b):
            return pltpu.make_async_copy(dst_hbm.at[wid, j], idx_v.at[b],
                                         isem.at[b])

        def sc_wait(b):
            return pltpu.make_async_copy(ebuf[b], sh_agg.at[idx_v.at[b]],
                                         ssem.at[b])

        for b in range(_NBUF):
            rd(b, b).start()
            rix(b, b).start()

        def body(g, carry):
            for b in range(_NBUF):
                j = g * _NBUF + b
                rd(j, b).wait()
                rix(j, b).wait()
                pltpu.async_copy(ebuf[b], sh_agg.at[idx_v.at[b]], ssem.at[b],
                                 add=True)
            for b in range(_NBUF):
                j = g * _NBUF + b

                @pl.when(g < ngroups - 1)
                def _(j=j, b=b):
                    sc_wait(b).wait()
                    rd(j + _NBUF, b).start()
                    rix(j + _NBUF, b).start()
            return carry

        lax.fori_loop(0, ngroups, body, 0)
        for b in range(_NBUF):
            sc_wait(b).wait()

        plsc.subcore_barrier()

        @pl.when(s == 0)
        def _():
            pltpu.sync_copy(sh_agg, aggp_hbm.at[c])

    return k(e_aug, dst3, zeros2d)


def kernel(x, edge_attr, state, params, edge_index, batch, bond_batch):
    N, DN = x.shape
    E, DE = edge_attr.shape
    G, DS = state.shape
    EMB = params['pre_v'][-1][0].shape[1]
    H = 2 * EMB

    (wv1, bv1), (wv2, bv2) = params['pre_v']
    (we1, be1), (we2, be2) = params['pre_e']
    (wu1, bu1), (wu2, bu2) = params['pre_u']
    (pe1, pe1b), (pe2, pe2b), (pe3, pe3b) = params['phi_e']
    (pv1, pv1b), (pv2, pv2b), (pv3, pv3b) = params['phi_v']
    (pu1, pu1b), (pu2, pu2b), (pu3, pu3b) = params['phi_u']

    w_dst, w_src, w_he, w_hub = (pe1[0:EMB], pe1[EMB:2 * EMB],
                                 pe1[2 * EMB:3 * EMB], pe1[3 * EMB:4 * EMB])
    v_agg, v_hx, v_hub = pv1[0:EMB], pv1[EMB:2 * EMB], pv1[2 * EMB:3 * EMB]
    u_ue, u_uv, u_hu = pu1[0:EMB], pu1[EMB:2 * EMB], pu1[2 * EMB:3 * EMB]

    row = lambda v: v.reshape(1, -1)

    TN = 1000
    assert N % TN == 0
    h_x, pd, ps = pl.pallas_call(
        _pre_v_body,
        grid=(N // TN,),
        in_specs=[pl.BlockSpec((TN, DN), lambda i: (i, 0)),
                  pl.BlockSpec((DN, H), lambda i: (0, 0)),
                  pl.BlockSpec((1, H), lambda i: (0, 0)),
                  pl.BlockSpec((H, EMB), lambda i: (0, 0)),
                  pl.BlockSpec((1, EMB), lambda i: (0, 0)),
                  pl.BlockSpec((EMB, H), lambda i: (0, 0)),
                  pl.BlockSpec((EMB, H), lambda i: (0, 0))],
        out_specs=[pl.BlockSpec((TN, EMB), lambda i: (i, 0)),
                   pl.BlockSpec((TN, H), lambda i: (i, 0)),
                   pl.BlockSpec((TN, H), lambda i: (i, 0))],
        out_shape=[jax.ShapeDtypeStruct((N, EMB), _F32),
                   jax.ShapeDtypeStruct((N, H), _F32),
                   jax.ShapeDtypeStruct((N, H), _F32)],
    )(x, wv1, row(bv1), wv2, row(bv2), w_dst, w_src)

    h_u, table_ue, table_uv = pl.pallas_call(
        _graph_prep_body,
        out_shape=[jax.ShapeDtypeStruct((G, EMB), _F32),
                   jax.ShapeDtypeStruct((G, H), _F32),
                   jax.ShapeDtypeStruct((G, H), _F32)],
    )(state, wu1, row(bu1), wu2, row(bu2), w_hub, row(pe1b), v_hub, row(pv1b))

    TE = 2000
    EH = E // 2
    assert EH % (_NTILES * _CHUNK * _NBUF) == 0 and EH % TE == 0
    nchunk = EH // (_NTILES * _CHUNK)
    we1b, we2b = we1.astype(jnp.bfloat16), we2.astype(jnp.bfloat16)
    w_heb = w_he.astype(jnp.bfloat16)
    pe2b_, pe3b_ = pe2.astype(jnp.bfloat16), pe3.astype(jnp.bfloat16)
    ueaccs, aggps = [], []
    e_out = None
    for hh in range(2):
        sl = slice(hh * EH, (hh + 1) * EH)
        dst_h = edge_index[1, sl]
        src_h = edge_index[0, sl]
        dst2 = dst_h.reshape(_NTILES, EH // _NTILES)
        src2 = src_h.reshape(_NTILES, EH // _NTILES)
        dst3 = dst_h.reshape(_NTILES, nchunk, _CHUNK)
        rd, rs = _sc_gather(pd, ps, dst2, src2, EH, H)
        bb3 = bond_batch[sl].reshape(EH // TE, 1, TE)
        off = hh * (EH // TE)
        alias_specs = ([] if hh == 0
                       else [pl.BlockSpec(memory_space=pl.ANY)])
        alias_args = [] if hh == 0 else [e_out]
        e_aug, e_out, ueacc_h = pl.pallas_call(
            functools.partial(_edge_body, TE, G, hh),
            grid=(EH // TE,),
            in_specs=([pl.BlockSpec((TE, H), lambda i: (i, 0)),
                       pl.BlockSpec((TE, H), lambda i: (i, 0)),
                       pl.BlockSpec((TE, DE), lambda i: (i, 0)),
                       pl.BlockSpec((1, 1, TE), lambda i: (i, 0, 0))]
                      + alias_specs
                      + [pl.BlockSpec((DE, H), lambda i: (0, 0)),
                         pl.BlockSpec((1, H), lambda i: (0, 0)),
                         pl.BlockSpec((H, EMB), lambda i: (0, 0)),
                         pl.BlockSpec((1, EMB), lambda i: (0, 0)),
                         pl.BlockSpec((EMB, H), lambda i: (0, 0)),
                         pl.BlockSpec((G, H), lambda i: (0, 0)),
                         pl.BlockSpec((H, H), lambda i: (0, 0)),
                         pl.BlockSpec((1, H), lambda i: (0, 0)),
                         pl.BlockSpec((H, EMB), lambda i: (0, 0)),
                         pl.BlockSpec((1, EMB), lambda i: (0, 0))]),
            out_specs=[pl.BlockSpec((TE, H), lambda i: (i, 0)),
                       pl.BlockSpec((TE, EMB), lambda i, off=off: (i + off, 0)),
                       pl.BlockSpec((G, EMB + 1), lambda i: (0, 0))],
            out_shape=[jax.ShapeDtypeStruct((EH, H), _F32),
                       jax.ShapeDtypeStruct((E, EMB), _F32),
                       jax.ShapeDtypeStruct((G, EMB + 1), _F32)],
            input_output_aliases=({} if hh == 0 else {4: 1}),
        )(rd, rs, edge_attr[sl], bb3, *alias_args,
          we1b, row(be1), we2b, row(be2), w_heb,
          table_ue.astype(jnp.bfloat16),
          pe2b_, row(pe2b), pe3b_, row(pe3b))
        ueaccs.append(ueacc_h)
        aggps.append(_sc_scatter(e_aug, dst3,
                                 jnp.zeros((_CHUNK, H), _F32), N, H))

    b3 = batch.reshape(N // TN, 1, TN)
    x_out, uvacc = pl.pallas_call(
        functools.partial(_node_body, TN, G, EMB),
        grid=(N // TN,),
        in_specs=[pl.BlockSpec((TN, H), lambda i: (i, 0)),
                  pl.BlockSpec((TN, H), lambda i: (i, 0)),
                  pl.BlockSpec((TN, H), lambda i: (i, 0)),
                  pl.BlockSpec((TN, H), lambda i: (i, 0)),
                  pl.BlockSpec((TN, EMB), lambda i: (i, 0)),
                  pl.BlockSpec((1, 1, TN), lambda i: (i, 0, 0)),
                  pl.BlockSpec((EMB, H), lambda i: (0, 0)),
                  pl.BlockSpec((EMB, H), lambda i: (0, 0)),
                  pl.BlockSpec((G, H), lambda i: (0, 0)),
                  pl.BlockSpec((H, H), lambda i: (0, 0)),
                  pl.BlockSpec((1, H), lambda i: (0, 0)),
                  pl.BlockSpec((H, EMB), lambda i: (0, 0)),
                  pl.BlockSpec((1, EMB), lambda i: (0, 0))],
        out_specs=[pl.BlockSpec((TN, EMB), lambda i: (i, 0)),
                   pl.BlockSpec((G, EMB + 1), lambda i: (0, 0))],
        out_shape=[jax.ShapeDtypeStruct((N, EMB), _F32),
                   jax.ShapeDtypeStruct((G, EMB + 1), _F32)],
    )(aggps[0][0], aggps[0][1], aggps[1][0], aggps[1][1], h_x, b3,
      v_agg, v_hx, table_uv, pv2, row(pv2b), pv3, row(pv3b))

    x_final, u_out = pl.pallas_call(
        functools.partial(_final_body, TN, G, EMB),
        grid=(N // TN,),
        in_specs=[pl.BlockSpec((G, EMB), lambda i: (0, 0)),
                  pl.BlockSpec((1, 1, TN), lambda i: (i, 0, 0)),
                  pl.BlockSpec((G, EMB + 1), lambda i: (0, 0)),
                  pl.BlockSpec((G, EMB + 1), lambda i: (0, 0)),
                  pl.BlockSpec((G, EMB + 1), lambda i: (0, 0)),
                  pl.BlockSpec((G, EMB), lambda i: (0, 0)),
                  pl.BlockSpec((EMB, H), lambda i: (0, 0)),
                  pl.BlockSpec((EMB, H), lambda i: (0, 0)),
                  pl.BlockSpec((EMB, H), lambda i: (0, 0)),
                  pl.BlockSpec((1, H), lambda i: (0, 0)),
                  pl.BlockSpec((H, H), lambda i: (0, 0)),
                  pl.BlockSpec((1, H), lambda i: (0, 0)),
                  pl.BlockSpec((H, EMB), lambda i: (0, 0)),
                  pl.BlockSpec((1, EMB), lambda i: (0, 0))],
        out_specs=[pl.BlockSpec((TN, EMB), lambda i: (i, 0)),
                   pl.BlockSpec((G, EMB), lambda i: (0, 0))],
        out_shape=[jax.ShapeDtypeStruct((N, EMB), _F32),
                   jax.ShapeDtypeStruct((G, EMB), _F32)],
    )(x_out[:G], b3, ueaccs[0], ueaccs[1], uvacc, h_u,
      u_ue, u_uv, u_hu, row(pu1b), pu2, row(pu2b), pu3, row(pu3b))

    return (x_final, e_out, u_out)


# speedup vs baseline: 1.2437x; 1.1900x over previous
import functools

import jax
import jax.numpy as jnp
from jax import lax
from jax.experimental import pallas as pl
from jax.experimental.pallas import tpu as pltpu
from jax.experimental.pallas import tpu_sc as plsc

_LOG2 = 0.6931471805599453
_F32 = jnp.float32


def _ssp(t):
    return jnp.maximum(t, 0.0) + jnp.log(1.0 + jnp.exp(-jnp.abs(t))) - _LOG2


def _dot(a, b):
    return jnp.dot(a, b, preferred_element_type=_F32)


def _pre_v_body(x_ref, w1, b1, w2, b2, wdst, wsrc, o_ref, pd_ref, ps_ref):
    h = _ssp(_dot(x_ref[...], w1[...]) + b1[...])
    hx = _ssp(_dot(h, w2[...]) + b2[...])
    o_ref[...] = hx
    pd_ref[...] = _dot(hx, wdst[...])
    ps_ref[...] = _dot(hx, wsrc[...])


def _graph_prep_body(s_ref, w1, b1, w2, b2, wd, be1, vc, bv1,
                     hu_ref, ue_ref, uv_ref):
    h = _ssp(_dot(s_ref[...], w1[...]) + b1[...])
    hu = _ssp(_dot(h, w2[...]) + b2[...])
    hu_ref[...] = hu
    ue_ref[...] = _dot(hu, wd[...]) + be1[...]
    uv_ref[...] = _dot(hu, vc[...]) + bv1[...]


def _edge_body(T, G, half,
               rd_ref, rs_ref, ea_ref, bb_ref, *rest):
    (we1, be1, we2, be2, wc, ue, v2, c2, v3, c3,
     e_ref, eo_ref, uacc_ref) = rest[1 if half else 0:]
    EMB = eo_ref.shape[0]
    bf16 = jnp.bfloat16
    t1 = _ssp(_dot(ea_ref[...].astype(bf16), we1[...]) + be1[...])
    he = _ssp(_dot(t1.astype(bf16), we2[...]) + be2[...])
    bb = bb_ref[0, 0, :]
    onehot = (bb[:, None] == lax.broadcasted_iota(jnp.int32, (T, G), 1)
              ).astype(bf16)
    z = (rd_ref[...].astype(_F32) + rs_ref[...].astype(_F32)
         + _dot(he.astype(bf16), wc[...]) + _dot(onehot, ue[...]))
    z1 = _ssp(z)
    z2 = _ssp(_dot(z1.astype(bf16), v2[...]) + c2[...])
    e = _ssp(_dot(z2.astype(bf16), v3[...]) + c3[...])
    e_ref[...] = jnp.concatenate(
        [e, jnp.ones((T, 1), _F32), jnp.zeros((T, 63), _F32)], axis=1)
    eye = (lax.broadcasted_iota(jnp.int32, (EMB, EMB), 0)
           == lax.broadcasted_iota(jnp.int32, (EMB, EMB), 1)).astype(_F32)
    eo_ref[...] = lax.dot_general(eye, e + he, (((1,), (1,)), ((), ())),
                                  preferred_element_type=_F32)

    onehot_t = (bb[None, :] == lax.broadcasted_iota(jnp.int32, (G, T), 0)
                ).astype(_F32)
    eaug = jnp.concatenate([e, jnp.ones((T, 1), _F32)], axis=1)

    @pl.when(pl.program_id(0) == 0)
    def _():
        uacc_ref[...] = jnp.zeros_like(uacc_ref)

    uacc_ref[...] += _dot(onehot_t, eaug)


def _node_body(TN, G, EMB,
               p00_ref, p01_ref, p10_ref, p11_ref, hx_ref, b_ref,
               va, vb, uv, v2, c2, v3, c3,
               xo_ref, uvacc_ref):
    p = (p00_ref[...] + p01_ref[...]) + (p10_ref[...] + p11_ref[...])
    cnt = p[:, EMB:EMB + 1]
    agg = p[:, :EMB] / jnp.maximum(cnt, 1.0)
    b = b_ref[0, 0, :]
    onehot = (b[:, None] == lax.broadcasted_iota(jnp.int32, (TN, G), 1)
              ).astype(_F32)
    hx = hx_ref[...]
    z1 = _ssp(_dot(agg, va[...]) + _dot(hx, vb[...]) + _dot(onehot, uv[...]))
    z2 = _ssp(_dot(z1, v2[...]) + c2[...])
    v = _ssp(_dot(z2, v3[...]) + c3[...])
    xo_ref[...] = v + hx

    onehot_t = (b[None, :] == lax.broadcasted_iota(jnp.int32, (G, TN), 0)
                ).astype(_F32)
    vaug = jnp.concatenate([v, jnp.ones((TN, 1), _F32)], axis=1)

    @pl.when(pl.program_id(0) == 0)
    def _():
        uvacc_ref[...] = jnp.zeros_like(uvacc_ref)

    uvacc_ref[...] += _dot(onehot_t, vaug)


def _final_body(TN, G, EMB,
                x128_ref, b_ref, ueacc_ref, ueacc2_ref, uvacc_ref, hu_ref,
                u1e, u1v, u1h, bu1, u2, bu2, u3, bu3,
                xf_ref, uo_ref):
    @pl.when(pl.program_id(0) == 0)
    def _():
        uea = ueacc_ref[...] + ueacc2_ref[...]
        ue_m = uea[:, :EMB] / jnp.maximum(uea[:, EMB:EMB + 1], 1.0)
        uv_m = uvacc_ref[:, :EMB] / jnp.maximum(uvacc_ref[:, EMB:EMB + 1], 1.0)
        hu = hu_ref[...]
        z1 = _ssp(_dot(ue_m, u1e[...]) + _dot(uv_m, u1v[...])
                  + _dot(hu, u1h[...]) + bu1[...])
        z2 = _ssp(_dot(z1, u2[...]) + bu2[...])
        u = _ssp(_dot(z2, u3[...]) + bu3[...])
        uo_ref[...] = u + hu

    b = b_ref[0, 0, :]
    onehot = (b[:, None] == lax.broadcasted_iota(jnp.int32, (TN, G), 1)
              ).astype(_F32)
    xf_ref[...] = _dot(onehot, x128_ref[...])


_NTILES = 32
_CHUNK = 40
_NBUF = 5


def _sc_gather(pd, ps, dst2, src2, E, D):
    per_tile = dst2.shape[1]
    nchunk = per_tile // _CHUNK
    ngroups = nchunk // _NBUF
    mesh = plsc.VectorSubcoreMesh(core_axis_name="c", subcore_axis_name="s")

    @functools.partial(
        pl.kernel, mesh=mesh,
        out_type=[jax.ShapeDtypeStruct((E, D), _F32),
                  jax.ShapeDtypeStruct((E, D), _F32)],
        scratch_types=([pltpu.VMEM((per_tile,), jnp.int32),
                        pltpu.VMEM((per_tile,), jnp.int32)]
                       + [pltpu.VMEM((_CHUNK, D), _F32)] * (2 * _NBUF)
                       + [pltpu.SemaphoreType.DMA((_NBUF,))] * 4),
    )
    def k(pd_hbm, ps_hbm, dst_hbm, src_hbm, rd_hbm, rs_hbm,
          idx_d, idx_s, *bufs_and_sems):
        bd = bufs_and_sems[:_NBUF]
        bs = bufs_and_sems[_NBUF:2 * _NBUF]
        gsd, gss, wsd, wss = bufs_and_sems[2 * _NBUF:]
        c = lax.axis_index("c")
        s = lax.axis_index("s")
        wid = s * 2 + c
        base = wid * per_tile
        pltpu.sync_copy(dst_hbm.at[wid], idx_d)
        pltpu.sync_copy(src_hbm.at[wid], idx_s)

        def g_d(j, b):
            return pltpu.make_async_copy(
                pd_hbm.at[idx_d.at[pl.ds(j * _CHUNK, _CHUNK)]], bd[b],
                gsd.at[b])

        def g_s(j, b):
            return pltpu.make_async_copy(
                ps_hbm.at[idx_s.at[pl.ds(j * _CHUNK, _CHUNK)]], bs[b],
                gss.at[b])

        def w_d(j, b):
            return pltpu.make_async_copy(
                bd[b], rd_hbm.at[pl.ds(base + j * _CHUNK, _CHUNK)], wsd.at[b])

        def w_s(j, b):
            return pltpu.make_async_copy(
                bs[b], rs_hbm.at[pl.ds(base + j * _CHUNK, _CHUNK)], wss.at[b])

        for b in range(_NBUF):
            g_d(b, b).start()
            g_s(b, b).start()

        def body(g, carry):
            for b in range(_NBUF):
                j = g * _NBUF + b
                g_d(j, b).wait()
                g_s(j, b).wait()
                w_d(j, b).start()
                w_s(j, b).start()
            for b in range(_NBUF):
                j = g * _NBUF + b

                @pl.when(g < ngroups - 1)
                def _(j=j, b=b):
                    w_d(j, b).wait()
                    w_s(j, b).wait()
                    g_d(j + _NBUF, b).start()
                    g_s(j + _NBUF, b).start()
            return carry

        lax.fori_loop(0, ngroups, body, 0)
        jlast = (ngroups - 1) * _NBUF
        for b in range(_NBUF):
            w_d(jlast + b, b).wait()
            w_s(jlast + b, b).wait()

    return k(pd, ps, dst2, src2)


def _sc_scatter(e_aug, dst3, zeros2d, N, D):
    nchunk = dst3.shape[1]
    per_tile = nchunk * _CHUNK
    ngroups = nchunk // _NBUF
    mesh = plsc.VectorSubcoreMesh(core_axis_name="c", subcore_axis_name="s")

    @functools.partial(
        pl.kernel, mesh=mesh,
        out_type=jax.ShapeDtypeStruct((2, N, D), _F32),
        scratch_types=([pltpu.VMEM((_NBUF, _CHUNK), jnp.int32)]
                       + [pltpu.VMEM((_CHUNK, D), _F32)] * _NBUF
                       + [pltpu.SemaphoreType.DMA((_NBUF,))] * 3
                       + [pltpu.VMEM_SHARED((N, D), _F32)]),
    )
    def k(e_hbm, dst_hbm, z2_hbm, aggp_hbm, idx_v, *rest):
        ebuf = rest[:_NBUF]
        rsem, isem, ssem, sh_agg = rest[_NBUF:]
        c = lax.axis_index("c")
        s = lax.axis_index("s")
        wid = s * 2 + c
        base = wid * per_tile

        pltpu.sync_copy(z2_hbm, ebuf[0])
        nzc = N // _CHUNK

        def zbody(t, carry):
            m = s * 16 + t

            @pl.when(m < nzc)
            def _():
                pltpu.sync_copy(ebuf[0], sh_agg.at[pl.ds(m * _CHUNK, _CHUNK)])
            return carry

        lax.fori_loop(0, (nzc + 15) // 16, zbody, 0)
        plsc.subcore_barrier()

        def rd(j, b):
            return pltpu.make_async_copy(
                e_hbm.at[pl.ds(base + j * _CHUNK, _CHUNK)], ebuf[b],
                rsem.at[b])

        def rix(j, b):
            return pltpu.make_async_copy(dst_hbm.at[wid, j], idx_v.at[b],
                                         isem.at[b])

        def sc_wait(b):
            return pltpu.make_async_copy(ebuf[b], sh_agg.at[idx_v.at[b]],
                                         ssem.at[b])

        for b in range(_NBUF):
            rd(b, b).start()
            rix(b, b).start()

        def body(g, carry):
            for b in range(_NBUF):
                j = g * _NBUF + b
                rd(j, b).wait()
                rix(j, b).wait()
                pltpu.async_copy(ebuf[b], sh_agg.at[idx_v.at[b]], ssem.at[b],
                                 add=True)
            for b in range(_NBUF):
                j = g * _NBUF + b

                @pl.when(g < ngroups - 1)
                def _(j=j, b=b):
                    sc_wait(b).wait()
                    rd(j + _NBUF, b).start()
                    rix(j + _NBUF, b).start()
            return carry

        lax.fori_loop(0, ngroups, body, 0)
        for b in range(_NBUF):
            sc_wait(b).wait()

        plsc.subcore_barrier()

        @pl.when(s == 0)
        def _():
            pltpu.sync_copy(sh_agg, aggp_hbm.at[c])

    return k(e_aug, dst3, zeros2d)


def kernel(x, edge_attr, state, params, edge_index, batch, bond_batch):
    N, DN = x.shape
    E, DE = edge_attr.shape
    G, DS = state.shape
    EMB = params['pre_v'][-1][0].shape[1]
    H = 2 * EMB

    (wv1, bv1), (wv2, bv2) = params['pre_v']
    (we1, be1), (we2, be2) = params['pre_e']
    (wu1, bu1), (wu2, bu2) = params['pre_u']
    (pe1, pe1b), (pe2, pe2b), (pe3, pe3b) = params['phi_e']
    (pv1, pv1b), (pv2, pv2b), (pv3, pv3b) = params['phi_v']
    (pu1, pu1b), (pu2, pu2b), (pu3, pu3b) = params['phi_u']

    w_dst, w_src, w_he, w_hub = (pe1[0:EMB], pe1[EMB:2 * EMB],
                                 pe1[2 * EMB:3 * EMB], pe1[3 * EMB:4 * EMB])
    v_agg, v_hx, v_hub = pv1[0:EMB], pv1[EMB:2 * EMB], pv1[2 * EMB:3 * EMB]
    u_ue, u_uv, u_hu = pu1[0:EMB], pu1[EMB:2 * EMB], pu1[2 * EMB:3 * EMB]

    row = lambda v: v.reshape(1, -1)

    TN = 1000
    assert N % TN == 0
    h_x, pd, ps = pl.pallas_call(
        _pre_v_body,
        grid=(N // TN,),
        in_specs=[pl.BlockSpec((TN, DN), lambda i: (i, 0)),
                  pl.BlockSpec((DN, H), lambda i: (0, 0)),
                  pl.BlockSpec((1, H), lambda i: (0, 0)),
                  pl.BlockSpec((H, EMB), lambda i: (0, 0)),
                  pl.BlockSpec((1, EMB), lambda i: (0, 0)),
                  pl.BlockSpec((EMB, H), lambda i: (0, 0)),
                  pl.BlockSpec((EMB, H), lambda i: (0, 0))],
        out_specs=[pl.BlockSpec((TN, EMB), lambda i: (i, 0)),
                   pl.BlockSpec((TN, H), lambda i: (i, 0)),
                   pl.BlockSpec((TN, H), lambda i: (i, 0))],
        out_shape=[jax.ShapeDtypeStruct((N, EMB), _F32),
                   jax.ShapeDtypeStruct((N, H), _F32),
                   jax.ShapeDtypeStruct((N, H), _F32)],
    )(x, wv1, row(bv1), wv2, row(bv2), w_dst, w_src)

    h_u, table_ue, table_uv = pl.pallas_call(
        _graph_prep_body,
        out_shape=[jax.ShapeDtypeStruct((G, EMB), _F32),
                   jax.ShapeDtypeStruct((G, H), _F32),
                   jax.ShapeDtypeStruct((G, H), _F32)],
    )(state, wu1, row(bu1), wu2, row(bu2), w_hub, row(pe1b), v_hub, row(pv1b))

    TE = 3200
    EH = E // 2
    assert EH % (_NTILES * _CHUNK * _NBUF) == 0 and EH % TE == 0
    nchunk = EH // (_NTILES * _CHUNK)
    we1b, we2b = we1.astype(jnp.bfloat16), we2.astype(jnp.bfloat16)
    w_heb = w_he.astype(jnp.bfloat16)
    pe2b_, pe3b_ = pe2.astype(jnp.bfloat16), pe3.astype(jnp.bfloat16)
    ueaccs, aggps = [], []
    bb3 = bond_batch.reshape(E // TE, 1, TE)
    eo_t = None
    for hh in range(2):
        sl = slice(hh * EH, (hh + 1) * EH)
        dst_h = edge_index[1, sl]
        src_h = edge_index[0, sl]
        dst2 = dst_h.reshape(_NTILES, EH // _NTILES)
        src2 = src_h.reshape(_NTILES, EH // _NTILES)
        dst3 = dst_h.reshape(_NTILES, nchunk, _CHUNK)
        rd, rs = _sc_gather(pd, ps, dst2, src2, EH, H)
        off = hh * (EH // TE)
        alias_specs = ([] if hh == 0
                       else [pl.BlockSpec(memory_space=pl.ANY)])
        alias_args = [] if hh == 0 else [eo_t]
        e_aug, eo_t, ueacc_h = pl.pallas_call(
            functools.partial(_edge_body, TE, G, hh),
            grid=(EH // TE,),
            in_specs=([pl.BlockSpec((TE, H), lambda i: (i, 0)),
                       pl.BlockSpec((TE, H), lambda i: (i, 0)),
                       pl.BlockSpec((TE, DE), lambda i, off=off: (i + off, 0)),
                       pl.BlockSpec((1, 1, TE),
                                    lambda i, off=off: (i + off, 0, 0))]
                      + alias_specs
                      + [pl.BlockSpec((DE, H), lambda i: (0, 0)),
                         pl.BlockSpec((1, H), lambda i: (0, 0)),
                         pl.BlockSpec((H, EMB), lambda i: (0, 0)),
                         pl.BlockSpec((1, EMB), lambda i: (0, 0)),
                         pl.BlockSpec((EMB, H), lambda i: (0, 0)),
                         pl.BlockSpec((G, H), lambda i: (0, 0)),
                         pl.BlockSpec((H, H), lambda i: (0, 0)),
                         pl.BlockSpec((1, H), lambda i: (0, 0)),
                         pl.BlockSpec((H, EMB), lambda i: (0, 0)),
                         pl.BlockSpec((1, EMB), lambda i: (0, 0))]),
            out_specs=[pl.BlockSpec((TE, H), lambda i: (i, 0)),
                       pl.BlockSpec((EMB, TE), lambda i, off=off: (0, i + off)),
                       pl.BlockSpec((G, EMB + 1), lambda i: (0, 0))],
            out_shape=[jax.ShapeDtypeStruct((EH, H), _F32),
                       jax.ShapeDtypeStruct((EMB, E), _F32),
                       jax.ShapeDtypeStruct((G, EMB + 1), _F32)],
            input_output_aliases=({} if hh == 0 else {4: 1}),
        )(rd, rs, edge_attr, bb3, *alias_args,
          we1b, row(be1), we2b, row(be2), w_heb,
          table_ue.astype(jnp.bfloat16),
          pe2b_, row(pe2b), pe3b_, row(pe3b))
        ueaccs.append(ueacc_h)
        aggps.append(_sc_scatter(e_aug, dst3,
                                 jnp.zeros((_CHUNK, H), _F32), N, H))
    e_out = eo_t.T

    b3 = batch.reshape(N // TN, 1, TN)
    x_out, uvacc = pl.pallas_call(
        functools.partial(_node_body, TN, G, EMB),
        grid=(N // TN,),
        in_specs=[pl.BlockSpec((TN, H), lambda i: (i, 0)),
                  pl.BlockSpec((TN, H), lambda i: (i, 0)),
                  pl.BlockSpec((TN, H), lambda i: (i, 0)),
                  pl.BlockSpec((TN, H), lambda i: (i, 0)),
                  pl.BlockSpec((TN, EMB), lambda i: (i, 0)),
                  pl.BlockSpec((1, 1, TN), lambda i: (i, 0, 0)),
                  pl.BlockSpec((EMB, H), lambda i: (0, 0)),
                  pl.BlockSpec((EMB, H), lambda i: (0, 0)),
                  pl.BlockSpec((G, H), lambda i: (0, 0)),
                  pl.BlockSpec((H, H), lambda i: (0, 0)),
                  pl.BlockSpec((1, H), lambda i: (0, 0)),
                  pl.BlockSpec((H, EMB), lambda i: (0, 0)),
                  pl.BlockSpec((1, EMB), lambda i: (0, 0))],
        out_specs=[pl.BlockSpec((TN, EMB), lambda i: (i, 0)),
                   pl.BlockSpec((G, EMB + 1), lambda i: (0, 0))],
        out_shape=[jax.ShapeDtypeStruct((N, EMB), _F32),
                   jax.ShapeDtypeStruct((G, EMB + 1), _F32)],
    )(aggps[0][0], aggps[0][1], aggps[1][0], aggps[1][1], h_x, b3,
      v_agg, v_hx, table_uv, pv2, row(pv2b), pv3, row(pv3b))

    x_final, u_out = pl.pallas_call(
        functools.partial(_final_body, TN, G, EMB),
        grid=(N // TN,),
        in_specs=[pl.BlockSpec((G, EMB), lambda i: (0, 0)),
                  pl.BlockSpec((1, 1, TN), lambda i: (i, 0, 0)),
                  pl.BlockSpec((G, EMB + 1), lambda i: (0, 0)),
                  pl.BlockSpec((G, EMB + 1), lambda i: (0, 0)),
                  pl.BlockSpec((G, EMB + 1), lambda i: (0, 0)),
                  pl.BlockSpec((G, EMB), lambda i: (0, 0)),
                  pl.BlockSpec((EMB, H), lambda i: (0, 0)),
                  pl.BlockSpec((EMB, H), lambda i: (0, 0)),
                  pl.BlockSpec((EMB, H), lambda i: (0, 0)),
                  pl.BlockSpec((1, H), lambda i: (0, 0)),
                  pl.BlockSpec((H, H), lambda i: (0, 0)),
                  pl.BlockSpec((1, H), lambda i: (0, 0)),
                  pl.BlockSpec((H, EMB), lambda i: (0, 0)),
                  pl.BlockSpec((1, EMB), lambda i: (0, 0))],
        out_specs=[pl.BlockSpec((TN, EMB), lambda i: (i, 0)),
                   pl.BlockSpec((G, EMB), lambda i: (0, 0))],
        out_shape=[jax.ShapeDtypeStruct((N, EMB), _F32),
                   jax.ShapeDtypeStruct((G, EMB), _F32)],
    )(x_out[:G], b3, ueaccs[0], ueaccs[1], uvacc, h_u,
      u_ue, u_uv, u_hu, row(pu1b), pu2, row(pu2b), pu3, row(pu3b))

    return (x_final, e_out, u_out)


# speedup vs baseline: 1.2936x; 1.0401x over previous
import functools

import jax
import jax.numpy as jnp
from jax import lax
from jax.experimental import pallas as pl
from jax.experimental.pallas import tpu as pltpu
from jax.experimental.pallas import tpu_sc as plsc

_LOG2 = 0.6931471805599453
_F32 = jnp.float32


def _ssp(t):
    return jnp.maximum(t, 0.0) + jnp.log(1.0 + jnp.exp(-jnp.abs(t))) - _LOG2


def _dot(a, b):
    return jnp.dot(a, b, preferred_element_type=_F32)


def _pre_v_body(x_ref, w1, b1, w2, b2, wdst, wsrc, o_ref, pd_ref, ps_ref):
    h = _ssp(_dot(x_ref[...], w1[...]) + b1[...])
    hx = _ssp(_dot(h, w2[...]) + b2[...])
    o_ref[...] = hx
    pd_ref[...] = _dot(hx, wdst[...])
    ps_ref[...] = _dot(hx, wsrc[...])


def _graph_prep_body(s_ref, w1, b1, w2, b2, wd, be1, vc, bv1,
                     hu_ref, ue_ref, uv_ref):
    h = _ssp(_dot(s_ref[...], w1[...]) + b1[...])
    hu = _ssp(_dot(h, w2[...]) + b2[...])
    hu_ref[...] = hu
    ue_ref[...] = _dot(hu, wd[...]) + be1[...]
    uv_ref[...] = _dot(hu, vc[...]) + bv1[...]


def _edge_body(T, G, half,
               rd_ref, rs_ref, ea_ref, bb_ref, *rest):
    (we1, be1, we2, be2, wc, ue, v2, c2, v3, c3,
     e_ref, eo_ref, uacc_ref) = rest[1 if half else 0:]
    EMB = eo_ref.shape[0]
    bf16 = jnp.bfloat16
    t1 = _ssp(_dot(ea_ref[...].astype(bf16), we1[...]) + be1[...])
    he = _ssp(_dot(t1.astype(bf16), we2[...]) + be2[...])
    bb = bb_ref[0, 0, :]
    onehot = (bb[:, None] == lax.broadcasted_iota(jnp.int32, (T, G), 1)
              ).astype(bf16)
    heb = he.astype(bf16)
    z = (rd_ref[...].astype(_F32) + rs_ref[...].astype(_F32)
         + _dot(heb, wc[...]) + _dot(onehot, ue[...]))
    z1 = _ssp(z.astype(bf16))
    z2 = _ssp((_dot(z1, v2[...]) + c2[...]).astype(bf16))
    e = _ssp(_dot(z2, v3[...]) + c3[...])
    e_ref[...] = jnp.concatenate(
        [e, jnp.ones((T, 1), _F32), jnp.zeros((T, 63), _F32)], axis=1)
    eye = (lax.broadcasted_iota(jnp.int32, (EMB, EMB), 0)
           == lax.broadcasted_iota(jnp.int32, (EMB, EMB), 1)).astype(_F32)
    eo_ref[...] = lax.dot_general(eye, e + he, (((1,), (1,)), ((), ())),
                                  preferred_element_type=_F32)

    onehot_t = (bb[None, :] == lax.broadcasted_iota(jnp.int32, (G, T), 0)
                ).astype(_F32)
    eaug = jnp.concatenate([e, jnp.ones((T, 1), _F32)], axis=1)

    @pl.when(pl.program_id(0) == 0)
    def _():
        uacc_ref[...] = jnp.zeros_like(uacc_ref)

    uacc_ref[...] += _dot(onehot_t, eaug)


def _node_body(TN, G, EMB,
               p00_ref, p01_ref, p10_ref, p11_ref, hx_ref, b_ref,
               va, vb, uv, v2, c2, v3, c3,
               xo_ref, uvacc_ref):
    p = (p00_ref[...] + p01_ref[...]) + (p10_ref[...] + p11_ref[...])
    cnt = p[:, EMB:EMB + 1]
    agg = p[:, :EMB] / jnp.maximum(cnt, 1.0)
    b = b_ref[0, 0, :]
    onehot = (b[:, None] == lax.broadcasted_iota(jnp.int32, (TN, G), 1)
              ).astype(_F32)
    hx = hx_ref[...]
    z1 = _ssp(_dot(agg, va[...]) + _dot(hx, vb[...]) + _dot(onehot, uv[...]))
    z2 = _ssp(_dot(z1, v2[...]) + c2[...])
    v = _ssp(_dot(z2, v3[...]) + c3[...])
    xo_ref[...] = v + hx

    onehot_t = (b[None, :] == lax.broadcasted_iota(jnp.int32, (G, TN), 0)
                ).astype(_F32)
    vaug = jnp.concatenate([v, jnp.ones((TN, 1), _F32)], axis=1)

    @pl.when(pl.program_id(0) == 0)
    def _():
        uvacc_ref[...] = jnp.zeros_like(uvacc_ref)

    uvacc_ref[...] += _dot(onehot_t, vaug)


def _final_body(TN, G, EMB,
                x128_ref, b_ref, ueacc_ref, ueacc2_ref, uvacc_ref, hu_ref,
                u1e, u1v, u1h, bu1, u2, bu2, u3, bu3,
                xf_ref, uo_ref):
    @pl.when(pl.program_id(0) == 0)
    def _():
        uea = ueacc_ref[...] + ueacc2_ref[...]
        ue_m = uea[:, :EMB] / jnp.maximum(uea[:, EMB:EMB + 1], 1.0)
        uv_m = uvacc_ref[:, :EMB] / jnp.maximum(uvacc_ref[:, EMB:EMB + 1], 1.0)
        hu = hu_ref[...]
        z1 = _ssp(_dot(ue_m, u1e[...]) + _dot(uv_m, u1v[...])
                  + _dot(hu, u1h[...]) + bu1[...])
        z2 = _ssp(_dot(z1, u2[...]) + bu2[...])
        u = _ssp(_dot(z2, u3[...]) + bu3[...])
        uo_ref[...] = u + hu

    b = b_ref[0, 0, :]
    onehot = (b[:, None] == lax.broadcasted_iota(jnp.int32, (TN, G), 1)
              ).astype(_F32)
    xf_ref[...] = _dot(onehot, x128_ref[...])


_NTILES = 32
_CHUNK = 40
_NBUF = 5


def _sc_gather(pd, ps, dst2, src2, E, D):
    per_tile = dst2.shape[1]
    nchunk = per_tile // _CHUNK
    ngroups = nchunk // _NBUF
    mesh = plsc.VectorSubcoreMesh(core_axis_name="c", subcore_axis_name="s")

    @functools.partial(
        pl.kernel, mesh=mesh,
        out_type=[jax.ShapeDtypeStruct((E, D), _F32),
                  jax.ShapeDtypeStruct((E, D), _F32)],
        scratch_types=([pltpu.VMEM((per_tile,), jnp.int32),
                        pltpu.VMEM((per_tile,), jnp.int32)]
                       + [pltpu.VMEM((_CHUNK, D), _F32)] * (2 * _NBUF)
                       + [pltpu.SemaphoreType.DMA((_NBUF,))] * 4),
    )
    def k(pd_hbm, ps_hbm, dst_hbm, src_hbm, rd_hbm, rs_hbm,
          idx_d, idx_s, *bufs_and_sems):
        bd = bufs_and_sems[:_NBUF]
        bs = bufs_and_sems[_NBUF:2 * _NBUF]
        gsd, gss, wsd, wss = bufs_and_sems[2 * _NBUF:]
        c = lax.axis_index("c")
        s = lax.axis_index("s")
        wid = s * 2 + c
        base = wid * per_tile
        pltpu.sync_copy(dst_hbm.at[wid], idx_d)
        pltpu.sync_copy(src_hbm.at[wid], idx_s)

        def g_d(j, b):
            return pltpu.make_async_copy(
                pd_hbm.at[idx_d.at[pl.ds(j * _CHUNK, _CHUNK)]], bd[b],
                gsd.at[b])

        def g_s(j, b):
            return pltpu.make_async_copy(
                ps_hbm.at[idx_s.at[pl.ds(j * _CHUNK, _CHUNK)]], bs[b],
                gss.at[b])

        def w_d(j, b):
            return pltpu.make_async_copy(
                bd[b], rd_hbm.at[pl.ds(base + j * _CHUNK, _CHUNK)], wsd.at[b])

        def w_s(j, b):
            return pltpu.make_async_copy(
                bs[b], rs_hbm.at[pl.ds(base + j * _CHUNK, _CHUNK)], wss.at[b])

        for b in range(_NBUF):
            g_d(b, b).start()
            g_s(b, b).start()

        def body(g, carry):
            for b in range(_NBUF):
                j = g * _NBUF + b
                g_d(j, b).wait()
                g_s(j, b).wait()
                w_d(j, b).start()
                w_s(j, b).start()
            for b in range(_NBUF):
                j = g * _NBUF + b

                @pl.when(g < ngroups - 1)
                def _(j=j, b=b):
                    w_d(j, b).wait()
                    w_s(j, b).wait()
                    g_d(j + _NBUF, b).start()
                    g_s(j + _NBUF, b).start()
            return carry

        lax.fori_loop(0, ngroups, body, 0)
        jlast = (ngroups - 1) * _NBUF
        for b in range(_NBUF):
            w_d(jlast + b, b).wait()
            w_s(jlast + b, b).wait()

    return k(pd, ps, dst2, src2)


def _sc_scatter(e_aug, dst3, zeros2d, N, D):
    nchunk = dst3.shape[1]
    per_tile = nchunk * _CHUNK
    ngroups = nchunk // _NBUF
    mesh = plsc.VectorSubcoreMesh(core_axis_name="c", subcore_axis_name="s")

    @functools.partial(
        pl.kernel, mesh=mesh,
        out_type=jax.ShapeDtypeStruct((2, N, D), _F32),
        scratch_types=([pltpu.VMEM((_NBUF, _CHUNK), jnp.int32)]
                       + [pltpu.VMEM((_CHUNK, D), _F32)] * _NBUF
                       + [pltpu.SemaphoreType.DMA((_NBUF,))] * 3
                       + [pltpu.VMEM_SHARED((N, D), _F32)]),
    )
    def k(e_hbm, dst_hbm, z2_hbm, aggp_hbm, idx_v, *rest):
        ebuf = rest[:_NBUF]
        rsem, isem, ssem, sh_agg = rest[_NBUF:]
        c = lax.axis_index("c")
        s = lax.axis_index("s")
        wid = s * 2 + c
        base = wid * per_tile

        pltpu.sync_copy(z2_hbm, ebuf[0])
        nzc = N // _CHUNK

        def zbody(t, carry):
            m = s * 16 + t

            @pl.when(m < nzc)
            def _():
                pltpu.sync_copy(ebuf[0], sh_agg.at[pl.ds(m * _CHUNK, _CHUNK)])
            return carry

        lax.fori_loop(0, (nzc + 15) // 16, zbody, 0)
        plsc.subcore_barrier()

        def rd(j, b):
            return pltpu.make_async_copy(
                e_hbm.at[pl.ds(base + j * _CHUNK, _CHUNK)], ebuf[b],
                rsem.at[b])

        def rix(j, b):
            return pltpu.make_async_copy(dst_hbm.at[wid, j], idx_v.at[b],
                                         isem.at[b])

        def sc_wait(b):
            return pltpu.make_async_copy(ebuf[b], sh_agg.at[idx_v.at[b]],
                                         ssem.at[b])

        for b in range(_NBUF):
            rd(b, b).start()
            rix(b, b).start()

        def body(g, carry):
            for b in range(_NBUF):
                j = g * _NBUF + b
                rd(j, b).wait()
                rix(j, b).wait()
                pltpu.async_copy(ebuf[b], sh_agg.at[idx_v.at[b]], ssem.at[b],
                                 add=True)
            for b in range(_NBUF):
                j = g * _NBUF + b

                @pl.when(g < ngroups - 1)
                def _(j=j, b=b):
                    sc_wait(b).wait()
                    rd(j + _NBUF, b).start()
                    rix(j + _NBUF, b).start()
            return carry

        lax.fori_loop(0, ngroups, body, 0)
        for b in range(_NBUF):
            sc_wait(b).wait()

        plsc.subcore_barrier()

        @pl.when(s == 0)
        def _():
            pltpu.sync_copy(sh_agg, aggp_hbm.at[c])

    return k(e_aug, dst3, zeros2d)


def kernel(x, edge_attr, state, params, edge_index, batch, bond_batch):
    N, DN = x.shape
    E, DE = edge_attr.shape
    G, DS = state.shape
    EMB = params['pre_v'][-1][0].shape[1]
    H = 2 * EMB

    (wv1, bv1), (wv2, bv2) = params['pre_v']
    (we1, be1), (we2, be2) = params['pre_e']
    (wu1, bu1), (wu2, bu2) = params['pre_u']
    (pe1, pe1b), (pe2, pe2b), (pe3, pe3b) = params['phi_e']
    (pv1, pv1b), (pv2, pv2b), (pv3, pv3b) = params['phi_v']
    (pu1, pu1b), (pu2, pu2b), (pu3, pu3b) = params['phi_u']

    w_dst, w_src, w_he, w_hub = (pe1[0:EMB], pe1[EMB:2 * EMB],
                                 pe1[2 * EMB:3 * EMB], pe1[3 * EMB:4 * EMB])
    v_agg, v_hx, v_hub = pv1[0:EMB], pv1[EMB:2 * EMB], pv1[2 * EMB:3 * EMB]
    u_ue, u_uv, u_hu = pu1[0:EMB], pu1[EMB:2 * EMB], pu1[2 * EMB:3 * EMB]

    row = lambda v: v.reshape(1, -1)

    TN = 1000
    assert N % TN == 0
    h_x, pd, ps = pl.pallas_call(
        _pre_v_body,
        grid=(N // TN,),
        in_specs=[pl.BlockSpec((TN, DN), lambda i: (i, 0)),
                  pl.BlockSpec((DN, H), lambda i: (0, 0)),
                  pl.BlockSpec((1, H), lambda i: (0, 0)),
                  pl.BlockSpec((H, EMB), lambda i: (0, 0)),
                  pl.BlockSpec((1, EMB), lambda i: (0, 0)),
                  pl.BlockSpec((EMB, H), lambda i: (0, 0)),
                  pl.BlockSpec((EMB, H), lambda i: (0, 0))],
        out_specs=[pl.BlockSpec((TN, EMB), lambda i: (i, 0)),
                   pl.BlockSpec((TN, H), lambda i: (i, 0)),
                   pl.BlockSpec((TN, H), lambda i: (i, 0))],
        out_shape=[jax.ShapeDtypeStruct((N, EMB), _F32),
                   jax.ShapeDtypeStruct((N, H), _F32),
                   jax.ShapeDtypeStruct((N, H), _F32)],
    )(x, wv1, row(bv1), wv2, row(bv2), w_dst, w_src)

    h_u, table_ue, table_uv = pl.pallas_call(
        _graph_prep_body,
        out_shape=[jax.ShapeDtypeStruct((G, EMB), _F32),
                   jax.ShapeDtypeStruct((G, H), _F32),
                   jax.ShapeDtypeStruct((G, H), _F32)],
    )(state, wu1, row(bu1), wu2, row(bu2), w_hub, row(pe1b), v_hub, row(pv1b))

    TE = 3200
    EH = E // 2
    assert EH % (_NTILES * _CHUNK * _NBUF) == 0 and EH % TE == 0
    nchunk = EH // (_NTILES * _CHUNK)
    we1b, we2b = we1.astype(jnp.bfloat16), we2.astype(jnp.bfloat16)
    w_heb = w_he.astype(jnp.bfloat16)
    pe2b_, pe3b_ = pe2.astype(jnp.bfloat16), pe3.astype(jnp.bfloat16)
    ueaccs, aggps = [], []
    bb3 = bond_batch.reshape(E // TE, 1, TE)
    eo_t = None
    for hh in range(2):
        sl = slice(hh * EH, (hh + 1) * EH)
        dst_h = edge_index[1, sl]
        src_h = edge_index[0, sl]
        dst2 = dst_h.reshape(_NTILES, EH // _NTILES)
        src2 = src_h.reshape(_NTILES, EH // _NTILES)
        dst3 = dst_h.reshape(_NTILES, nchunk, _CHUNK)
        rd, rs = _sc_gather(pd, ps, dst2, src2, EH, H)
        off = hh * (EH // TE)
        alias_specs = ([] if hh == 0
                       else [pl.BlockSpec(memory_space=pl.ANY)])
        alias_args = [] if hh == 0 else [eo_t]
        e_aug, eo_t, ueacc_h = pl.pallas_call(
            functools.partial(_edge_body, TE, G, hh),
            grid=(EH // TE,),
            in_specs=([pl.BlockSpec((TE, H), lambda i: (i, 0)),
                       pl.BlockSpec((TE, H), lambda i: (i, 0)),
                       pl.BlockSpec((TE, DE), lambda i, off=off: (i + off, 0)),
                       pl.BlockSpec((1, 1, TE),
                                    lambda i, off=off: (i + off, 0, 0))]
                      + alias_specs
                      + [pl.BlockSpec((DE, H), lambda i: (0, 0)),
                         pl.BlockSpec((1, H), lambda i: (0, 0)),
                         pl.BlockSpec((H, EMB), lambda i: (0, 0)),
                         pl.BlockSpec((1, EMB), lambda i: (0, 0)),
                         pl.BlockSpec((EMB, H), lambda i: (0, 0)),
                         pl.BlockSpec((G, H), lambda i: (0, 0)),
                         pl.BlockSpec((H, H), lambda i: (0, 0)),
                         pl.BlockSpec((1, H), lambda i: (0, 0)),
                         pl.BlockSpec((H, EMB), lambda i: (0, 0)),
                         pl.BlockSpec((1, EMB), lambda i: (0, 0))]),
            out_specs=[pl.BlockSpec((TE, H), lambda i: (i, 0)),
                       pl.BlockSpec((EMB, TE), lambda i, off=off: (0, i + off)),
                       pl.BlockSpec((G, EMB + 1), lambda i: (0, 0))],
            out_shape=[jax.ShapeDtypeStruct((EH, H), _F32),
                       jax.ShapeDtypeStruct((EMB, E), _F32),
                       jax.ShapeDtypeStruct((G, EMB + 1), _F32)],
            input_output_aliases=({} if hh == 0 else {4: 1}),
        )(rd, rs, edge_attr, bb3, *alias_args,
          we1b, row(be1), we2b, row(be2), w_heb,
          table_ue.astype(jnp.bfloat16),
          pe2b_, row(pe2b), pe3b_, row(pe3b))
        ueaccs.append(ueacc_h)
        aggps.append(_sc_scatter(e_aug, dst3,
                                 jnp.zeros((_CHUNK, H), _F32), N, H))
    e_out = eo_t.T

    b3 = batch.reshape(N // TN, 1, TN)
    x_out, uvacc = pl.pallas_call(
        functools.partial(_node_body, TN, G, EMB),
        grid=(N // TN,),
        in_specs=[pl.BlockSpec((TN, H), lambda i: (i, 0)),
                  pl.BlockSpec((TN, H), lambda i: (i, 0)),
                  pl.BlockSpec((TN, H), lambda i: (i, 0)),
                  pl.BlockSpec((TN, H), lambda i: (i, 0)),
                  pl.BlockSpec((TN, EMB), lambda i: (i, 0)),
                  pl.BlockSpec((1, 1, TN), lambda i: (i, 0, 0)),
                  pl.BlockSpec((EMB, H), lambda i: (0, 0)),
                  pl.BlockSpec((EMB, H), lambda i: (0, 0)),
                  pl.BlockSpec((G, H), lambda i: (0, 0)),
                  pl.BlockSpec((H, H), lambda i: (0, 0)),
                  pl.BlockSpec((1, H), lambda i: (0, 0)),
                  pl.BlockSpec((H, EMB), lambda i: (0, 0)),
                  pl.BlockSpec((1, EMB), lambda i: (0, 0))],
        out_specs=[pl.BlockSpec((TN, EMB), lambda i: (i, 0)),
                   pl.BlockSpec((G, EMB + 1), lambda i: (0, 0))],
        out_shape=[jax.ShapeDtypeStruct((N, EMB), _F32),
                   jax.ShapeDtypeStruct((G, EMB + 1), _F32)],
    )(aggps[0][0], aggps[0][1], aggps[1][0], aggps[1][1], h_x, b3,
      v_agg, v_hx, table_uv, pv2, row(pv2b), pv3, row(pv3b))

    x_final, u_out = pl.pallas_call(
        functools.partial(_final_body, TN, G, EMB),
        grid=(N // TN,),
        in_specs=[pl.BlockSpec((G, EMB), lambda i: (0, 0)),
                  pl.BlockSpec((1, 1, TN), lambda i: (i, 0, 0)),
                  pl.BlockSpec((G, EMB + 1), lambda i: (0, 0)),
                  pl.BlockSpec((G, EMB + 1), lambda i: (0, 0)),
                  pl.BlockSpec((G, EMB + 1), lambda i: (0, 0)),
                  pl.BlockSpec((G, EMB), lambda i: (0, 0)),
                  pl.BlockSpec((EMB, H), lambda i: (0, 0)),
                  pl.BlockSpec((EMB, H), lambda i: (0, 0)),
                  pl.BlockSpec((EMB, H), lambda i: (0, 0)),
                  pl.BlockSpec((1, H), lambda i: (0, 0)),
                  pl.BlockSpec((H, H), lambda i: (0, 0)),
                  pl.BlockSpec((1, H), lambda i: (0, 0)),
                  pl.BlockSpec((H, EMB), lambda i: (0, 0)),
                  pl.BlockSpec((1, EMB), lambda i: (0, 0))],
        out_specs=[pl.BlockSpec((TN, EMB), lambda i: (i, 0)),
                   pl.BlockSpec((G, EMB), lambda i: (0, 0))],
        out_shape=[jax.ShapeDtypeStruct((N, EMB), _F32),
                   jax.ShapeDtypeStruct((G, EMB), _F32)],
    )(x_out[:G], b3, ueaccs[0], ueaccs[1], uvacc, h_u,
      u_ue, u_uv, u_hu, row(pu1b), pu2, row(pu2b), pu3, row(pu3b))

    return (x_final, e_out, u_out)
